# Initial kernel scaffold; baseline (speedup 1.0000x reference)
import jax
import jax.numpy as jnp
from jax.experimental import pallas as pl


def kernel(x, edge_index, batch, W1, b1, W2, b2, W3, b3, FW1, Fb1, FW2, Fb2):
    raise NotImplementedError("write your pallas kernel here")


# speedup vs baseline: 6.9035x; 6.9035x over previous
import functools

import jax
import jax.numpy as jnp
from jax import lax
from jax.experimental import pallas as pl
from jax.experimental.pallas import tpu as pltpu
from jax.experimental.pallas import tpu_sc as plsc

_N = 10000
_E = 320000
_G = 64
_F = 128
_NC = 2
_NS = 16
_EB = 80
_EPT = _E // _NS
_NEB = _EPT // _EB
_RB = 80
_NRB = _N // _RB
_BN = 2000


def _sc_mesh():
    return plsc.VectorSubcoreMesh(
        core_axis_name="c", subcore_axis_name="s", num_cores=_NC, num_subcores=_NS
    )


@functools.partial(
    pl.kernel,
    out_type=jax.ShapeDtypeStruct((_NC * _NS, _N), jnp.float32),
    mesh=_sc_mesh(),
    scratch_types=[
        pltpu.VMEM((_N,), jnp.float32),
        pltpu.VMEM((2000,), jnp.int32),
    ],
    compiler_params=pltpu.CompilerParams(needs_layout_passes=False),
)
def _deg_kernel(dst_hbm, out_hbm, deg_v, idx_v):
    core = lax.axis_index("c")
    s = lax.axis_index("s")
    wid = core * _NS + s
    zeros16 = jnp.zeros((16,), jnp.float32)

    def zero_body(i, carry):
        deg_v[pl.ds(i * 16, 16)] = zeros16
        return carry

    lax.fori_loop(0, _N // 16, zero_body, 0)

    ones16 = jnp.ones((16,), jnp.float32)
    ept = _E // (_NC * _NS)

    def blk_body(b, carry):
        pltpu.sync_copy(dst_hbm.at[pl.ds(wid * ept + b * 2000, 2000)], idx_v)

        def inner(j, c2):
            idx = idx_v[pl.ds(j * 16, 16)]
            plsc.addupdate_scatter(deg_v, [idx], ones16)
            return c2

        lax.fori_loop(0, 2000 // 16, inner, 0)
        return carry

    lax.fori_loop(0, ept // 2000, blk_body, 0)
    pltpu.sync_copy(deg_v, out_hbm.at[wid])


def _make_agg_kernel(n_chunks):
    cpc = n_chunks // _NC

    @functools.partial(
        pl.kernel,
        out_type=jax.ShapeDtypeStruct((n_chunks * _N, _F), jnp.float32),
        mesh=_sc_mesh(),
        scratch_types=[
            pltpu.VMEM((_EB,), jnp.int32),
            pltpu.VMEM((_EB,), jnp.int32),
            pltpu.VMEM((_EB, _F), jnp.float32),
            pltpu.VMEM((_RB, _F), jnp.float32),
            pltpu.VMEM_SHARED((_N, _F), jnp.float32),
            pltpu.SemaphoreType.DMA,
        ],
    )
    def agg_kernel(y_hbm, src_hbm, dst_hbm, out_hbm, src_v, dst_v, gbuf, zbuf, accum, sem):
        core = lax.axis_index("c")
        s = lax.axis_index("s")
        zeros16 = jnp.zeros((16,), jnp.float32)

        def zb_body(i, carry):
            r = i // (_F // 16)
            j = i % (_F // 16)
            zbuf[r, pl.ds(j * 16, 16)] = zeros16
            return carry

        lax.fori_loop(0, _RB * (_F // 16), zb_body, 0)

        nrb_mine = (_NRB - s + _NS - 1) // _NS

        for cc in range(cpc):
            chunk = core * cpc + cc
            off = chunk * _N

            def zero_body(k, carry):
                r0 = (s + k * _NS) * _RB
                pltpu.sync_copy(zbuf, accum.at[pl.ds(r0, _RB)])
                return carry

            lax.fori_loop(0, nrb_mine, zero_body, 0)
            plsc.subcore_barrier()

            def edge_body(b, carry):
                e0 = s * _EPT + b * _EB
                pltpu.sync_copy(src_hbm.at[pl.ds(e0, _EB)], src_v)
                pltpu.sync_copy(dst_hbm.at[pl.ds(e0, _EB)], dst_v)
                for j in range(_EB // 16):
                    src_v[pl.ds(j * 16, 16)] = src_v[pl.ds(j * 16, 16)] + off
                pltpu.async_copy(y_hbm.at[src_v], gbuf, sem).wait()
                pltpu.sync_copy(gbuf, accum.at[dst_v], add=True)
                return carry

            lax.fori_loop(0, _NEB, edge_body, 0)
            plsc.subcore_barrier()

            def wr_body(k, carry):
                r0 = (s + k * _NS) * _RB
                pltpu.sync_copy(accum.at[pl.ds(r0, _RB)], out_hbm.at[pl.ds(off + r0, _RB)])
                return carry

            lax.fori_loop(0, nrb_mine, wr_body, 0)
            if cc + 1 < cpc:
                plsc.subcore_barrier()

    return agg_kernel


_agg2 = _make_agg_kernel(2)
_agg4 = _make_agg_kernel(4)


def _y1_call(x, W1, deg32):
    def body(x_ref, w_ref, deg_ref, y_ref, dinv_ref):
        d = jnp.sum(deg_ref[...], axis=1) + 2.0
        dinv = lax.rsqrt(d)[:, None]
        xw = jnp.dot(x_ref[...], w_ref[...], preferred_element_type=jnp.float32)
        y = xw * dinv
        y_ref[0] = y[:, :_F]
        y_ref[1] = y[:, _F:]
        dinv_ref[...] = dinv

    return pl.pallas_call(
        body,
        grid=(_N // _BN,),
        in_specs=[
            pl.BlockSpec((_BN, 128), lambda i: (i, 0)),
            pl.BlockSpec((128, 256), lambda i: (0, 0)),
            pl.BlockSpec((_BN, _NC * _NS), lambda i: (i, 0)),
        ],
        out_specs=[
            pl.BlockSpec((2, _BN, _F), lambda i: (0, i, 0)),
            pl.BlockSpec((_BN, 1), lambda i: (i, 0)),
        ],
        out_shape=[
            jax.ShapeDtypeStruct((2, _N, _F), jnp.float32),
            jax.ShapeDtypeStruct((_N, 1), jnp.float32),
        ],
    )(x, W1, deg32)


def _mid_call(agg, y, dinv, b, W, c_in, c_out):
    d_in = c_in * _F
    d_out = c_out * _F

    def body(agg_ref, y_ref, dinv_ref, b_ref, w_ref, out_ref):
        dv = dinv_ref[...]
        acc = jnp.zeros((_BN, d_out), jnp.float32)
        for c in range(c_in):
            h = dv * (agg_ref[c] + 2.0 * y_ref[c]) + b_ref[0, c * _F:(c + 1) * _F]
            h = jnp.maximum(h, 0.0)
            acc = acc + jnp.dot(
                h, w_ref[c * _F:(c + 1) * _F, :], preferred_element_type=jnp.float32
            )
        y_next = acc * dv
        for co in range(c_out):
            out_ref[co] = y_next[:, co * _F:(co + 1) * _F]

    return pl.pallas_call(
        body,
        grid=(_N // _BN,),
        in_specs=[
            pl.BlockSpec((c_in, _BN, _F), lambda i: (0, i, 0)),
            pl.BlockSpec((c_in, _BN, _F), lambda i: (0, i, 0)),
            pl.BlockSpec((_BN, 1), lambda i: (i, 0)),
            pl.BlockSpec((1, d_in), lambda i: (0, 0)),
            pl.BlockSpec((d_in, d_out), lambda i: (0, 0)),
        ],
        out_specs=pl.BlockSpec((c_out, _BN, _F), lambda i: (0, i, 0)),
        out_shape=jax.ShapeDtypeStruct((c_out, _N, _F), jnp.float32),
    )(agg, y, dinv, b, W)


def _head_call(agg3, y3, dinv, b3, batch2d, FW1, Fb1, FW2, Fb2):
    n_steps = _N // _BN

    def body(agg_ref, y_ref, dinv_ref, b_ref, bt_ref, fw1_ref, fb1_ref,
             fw2_ref, fb2_ref, out_ref, pooled):
        i = pl.program_id(0)
        dv = dinv_ref[...]
        h0 = dv * (agg_ref[0] + 2.0 * y_ref[0]) + b_ref[0, :_F]
        h1 = dv * (agg_ref[1] + 2.0 * y_ref[1]) + b_ref[0, _F:]
        h = jnp.concatenate([h0, h1], axis=1)
        bt = bt_ref[...]
        neg = jnp.float32(-jnp.inf)

        @pl.when(i == 0)
        def _():
            pooled[...] = jnp.full((_G, 2 * _F), neg, jnp.float32)

        segs = []
        for g in range(_G):
            m = bt == g
            segs.append(jnp.max(jnp.where(m, h, neg), axis=0))
        blockmax = jnp.stack(segs, axis=0)
        pooled[...] = jnp.maximum(pooled[...], blockmax)

        @pl.when(i == n_steps - 1)
        def _():
            p = pooled[...]
            p = jnp.where(p == neg, 0.0, p)
            z = jnp.dot(p, fw1_ref[...], preferred_element_type=jnp.float32) + fb1_ref[0]
            z = jnp.maximum(z, 0.0)
            out_ref[...] = (
                jnp.dot(z, fw2_ref[...], preferred_element_type=jnp.float32) + fb2_ref[0]
            )

    return pl.pallas_call(
        body,
        grid=(n_steps,),
        in_specs=[
            pl.BlockSpec((2, _BN, _F), lambda i: (0, i, 0)),
            pl.BlockSpec((2, _BN, _F), lambda i: (0, i, 0)),
            pl.BlockSpec((_BN, 1), lambda i: (i, 0)),
            pl.BlockSpec((1, 256), lambda i: (0, 0)),
            pl.BlockSpec((_BN, 1), lambda i: (i, 0)),
            pl.BlockSpec((256, 512), lambda i: (0, 0)),
            pl.BlockSpec((1, 512), lambda i: (0, 0)),
            pl.BlockSpec((512, 10), lambda i: (0, 0)),
            pl.BlockSpec((1, 10), lambda i: (0, 0)),
        ],
        out_specs=pl.BlockSpec((_G, 10), lambda i: (0, 0)),
        out_shape=jax.ShapeDtypeStruct((_G, 10), jnp.float32),
        scratch_shapes=[pltpu.VMEM((_G, 2 * _F), jnp.float32)],
    )(agg3, y3, dinv, b3, batch2d, FW1, Fb1, FW2, Fb2)


def kernel(x, edge_index, batch, W1, b1, W2, b2, W3, b3, FW1, Fb1, FW2, Fb2):
    src = edge_index[0].astype(jnp.int32)
    dst = edge_index[1].astype(jnp.int32)
    batch2d = batch.astype(jnp.int32).reshape(_N, 1)

    deg32 = _deg_kernel(dst)
    y1, dinv = _y1_call(x, W1, deg32.T)

    agg1 = _agg2(y1.reshape(2 * _N, _F), src, dst).reshape(2, _N, _F)
    y2 = _mid_call(agg1, y1, dinv, b1.reshape(1, 256), W2, 2, 4)

    agg2 = _agg4(y2.reshape(4 * _N, _F), src, dst).reshape(4, _N, _F)
    y3 = _mid_call(agg2, y2, dinv, b2.reshape(1, 512), W3, 4, 2)

    agg3 = _agg2(y3.reshape(2 * _N, _F), src, dst).reshape(2, _N, _F)
    return _head_call(
        agg3, y3, dinv, b3.reshape(1, 256), batch2d,
        FW1, Fb1.reshape(1, 512), FW2, Fb2.reshape(1, 10)
    )


# speedup vs baseline: 13.0336x; 1.8880x over previous
import functools

import jax
import jax.numpy as jnp
from jax import lax
from jax.experimental import pallas as pl
from jax.experimental.pallas import tpu as pltpu
from jax.experimental.pallas import tpu_sc as plsc

_N = 10000
_E = 320000
_G = 64
_F = 128
_NC = 2
_NS = 16
_EB = 80
_EPT = _E // _NS
_NEB = _EPT // _EB
_RB = 40
_NRB = _N // _RB
_BN = 2000


def _sc_mesh():
    return plsc.VectorSubcoreMesh(
        core_axis_name="c", subcore_axis_name="s", num_cores=_NC, num_subcores=_NS
    )


@functools.partial(
    pl.kernel,
    out_type=jax.ShapeDtypeStruct((_NC * _NS, _N), jnp.float32),
    mesh=_sc_mesh(),
    scratch_types=[
        pltpu.VMEM((_N,), jnp.float32),
        pltpu.VMEM((2000,), jnp.int32),
    ],
    compiler_params=pltpu.CompilerParams(needs_layout_passes=False),
)
def _deg_kernel(dst_hbm, out_hbm, deg_v, idx_v):
    core = lax.axis_index("c")
    s = lax.axis_index("s")
    wid = core * _NS + s
    zeros16 = jnp.zeros((16,), jnp.float32)

    def zero_body(i, carry):
        deg_v[pl.ds(i * 16, 16)] = zeros16
        return carry

    lax.fori_loop(0, _N // 16, zero_body, 0)

    ones16 = jnp.ones((16,), jnp.float32)
    ept = _E // (_NC * _NS)

    def blk_body(b, carry):
        pltpu.sync_copy(dst_hbm.at[pl.ds(wid * ept + b * 2000, 2000)], idx_v)

        def inner(j, c2):
            idx = idx_v[pl.ds(j * 16, 16)]
            plsc.addupdate_scatter(deg_v, [idx], ones16)
            return c2

        lax.fori_loop(0, 2000 // 16, inner, 0)
        return carry

    lax.fori_loop(0, ept // 2000, blk_body, 0)
    pltpu.sync_copy(deg_v, out_hbm.at[wid])


_K = 8
_NB = 4
_NGRP = _E // _EB // _K


def _make_agg_kernel(n_chunks):
    cpc = n_chunks // _NC

    @functools.partial(
        pl.kernel,
        out_type=jax.ShapeDtypeStruct((n_chunks * _N, _F), jnp.float32),
        mesh=_sc_mesh(),
        scratch_types=[
            pltpu.VMEM((_K, _EB), jnp.int32),
            pltpu.VMEM((_K, _EB), jnp.int32),
            [pltpu.VMEM((_EB, _F), jnp.float32) for _ in range(_NB)],
            pltpu.VMEM((_RB, _F), jnp.float32),
            pltpu.VMEM_SHARED((_N, _F), jnp.float32),
            pltpu.SemaphoreType.DMA((_NB,)),
            pltpu.SemaphoreType.DMA((_NB,)),
        ],
    )
    def agg_kernel(y_hbm, src_hbm, dst_hbm, out_hbm, src_g, dst_g, gbufs,
                   zbuf, accum, gsem, ssem):
        core = lax.axis_index("c")
        s = lax.axis_index("s")
        zeros16 = jnp.zeros((16,), jnp.float32)

        def zb_body(i, carry):
            r = i // (_F // 16)
            j = i % (_F // 16)
            zbuf[r, pl.ds(j * 16, 16)] = zeros16
            return carry

        lax.fori_loop(0, _RB * (_F // 16), zb_body, 0)

        nrb_mine = (_NRB - s + _NS - 1) // _NS

        for cc in range(cpc):
            chunk = core * cpc + cc
            off = chunk * _N

            def zero_body(k, carry):
                r0 = (s + k * _NS) * _RB
                pltpu.sync_copy(zbuf, accum.at[pl.ds(r0, _RB)])
                return carry

            lax.fori_loop(0, nrb_mine, zero_body, 0)
            plsc.subcore_barrier()

            def edge_body(g, carry):
                row0 = (s + g * _NS) * _K
                pltpu.sync_copy(src_hbm.at[pl.ds(row0, _K)], src_g)
                pltpu.sync_copy(dst_hbm.at[pl.ds(row0, _K)], dst_g)
                for k in range(_K):
                    for j in range(_EB // 16):
                        src_g[k, pl.ds(j * 16, 16)] = (
                            src_g[k, pl.ds(j * 16, 16)] + off
                        )
                gds = {}
                sds = {}
                for k in range(_NB):
                    gds[k] = pltpu.async_copy(
                        y_hbm.at[src_g.at[k]], gbufs[k], gsem.at[k]
                    )
                for k in range(_K):
                    gds[k].wait()
                    if k >= 2:
                        sds[k - 2].wait()
                        if k + 2 < _K:
                            gds[k + 2] = pltpu.async_copy(
                                y_hbm.at[src_g.at[k + 2]],
                                gbufs[(k + 2) % _NB],
                                gsem.at[(k + 2) % _NB],
                            )
                    sds[k] = pltpu.async_copy(
                        gbufs[k % _NB], accum.at[dst_g.at[k]],
                        ssem.at[k % _NB], add=True,
                    )
                sds[_K - 2].wait()
                sds[_K - 1].wait()
                return carry

            ngrp_mine = (_NGRP - s + _NS - 1) // _NS
            lax.fori_loop(0, ngrp_mine, edge_body, 0)
            plsc.subcore_barrier()

            def wr_body(k, carry):
                r0 = (s + k * _NS) * _RB
                pltpu.sync_copy(accum.at[pl.ds(r0, _RB)], out_hbm.at[pl.ds(off + r0, _RB)])
                return carry

            lax.fori_loop(0, nrb_mine, wr_body, 0)
            if cc + 1 < cpc:
                plsc.subcore_barrier()

    return agg_kernel


_agg2 = _make_agg_kernel(2)
_agg4 = _make_agg_kernel(4)


def _y1_call(x, W1, deg32):
    def body(x_ref, w_ref, deg_ref, y_ref, dinv_ref):
        d = jnp.sum(deg_ref[...], axis=1) + 2.0
        dinv = lax.rsqrt(d)[:, None]
        xw = jnp.dot(x_ref[...], w_ref[...], preferred_element_type=jnp.float32)
        y = xw * dinv
        y_ref[0] = y[:, :_F]
        y_ref[1] = y[:, _F:]
        dinv_ref[...] = dinv

    return pl.pallas_call(
        body,
        grid=(_N // _BN,),
        in_specs=[
            pl.BlockSpec((_BN, 128), lambda i: (i, 0)),
            pl.BlockSpec((128, 256), lambda i: (0, 0)),
            pl.BlockSpec((_BN, _NC * _NS), lambda i: (i, 0)),
        ],
        out_specs=[
            pl.BlockSpec((2, _BN, _F), lambda i: (0, i, 0)),
            pl.BlockSpec((_BN, 1), lambda i: (i, 0)),
        ],
        out_shape=[
            jax.ShapeDtypeStruct((2, _N, _F), jnp.float32),
            jax.ShapeDtypeStruct((_N, 1), jnp.float32),
        ],
    )(x, W1, deg32)


def _mid_call(agg, y, dinv, b, W, c_in, c_out):
    d_in = c_in * _F
    d_out = c_out * _F

    def body(agg_ref, y_ref, dinv_ref, b_ref, w_ref, out_ref):
        dv = dinv_ref[...]
        acc = jnp.zeros((_BN, d_out), jnp.float32)
        for c in range(c_in):
            h = dv * (agg_ref[c] + 2.0 * y_ref[c]) + b_ref[0, c * _F:(c + 1) * _F]
            h = jnp.maximum(h, 0.0)
            acc = acc + jnp.dot(
                h, w_ref[c * _F:(c + 1) * _F, :], preferred_element_type=jnp.float32
            )
        y_next = acc * dv
        for co in range(c_out):
            out_ref[co] = y_next[:, co * _F:(co + 1) * _F]

    return pl.pallas_call(
        body,
        grid=(_N // _BN,),
        in_specs=[
            pl.BlockSpec((c_in, _BN, _F), lambda i: (0, i, 0)),
            pl.BlockSpec((c_in, _BN, _F), lambda i: (0, i, 0)),
            pl.BlockSpec((_BN, 1), lambda i: (i, 0)),
            pl.BlockSpec((1, d_in), lambda i: (0, 0)),
            pl.BlockSpec((d_in, d_out), lambda i: (0, 0)),
        ],
        out_specs=pl.BlockSpec((c_out, _BN, _F), lambda i: (0, i, 0)),
        out_shape=jax.ShapeDtypeStruct((c_out, _N, _F), jnp.float32),
    )(agg, y, dinv, b, W)


def _head_call(agg3, y3, dinv, b3, batch2d, FW1, Fb1, FW2, Fb2):
    n_steps = _N // _BN

    def body(agg_ref, y_ref, dinv_ref, b_ref, bt_ref, fw1_ref, fb1_ref,
             fw2_ref, fb2_ref, out_ref, pooled):
        i = pl.program_id(0)
        dv = dinv_ref[...]
        h0 = dv * (agg_ref[0] + 2.0 * y_ref[0]) + b_ref[0, :_F]
        h1 = dv * (agg_ref[1] + 2.0 * y_ref[1]) + b_ref[0, _F:]
        h = jnp.concatenate([h0, h1], axis=1)
        bt = bt_ref[...]
        neg = jnp.float32(-jnp.inf)

        @pl.when(i == 0)
        def _():
            pooled[...] = jnp.full((_G, 2 * _F), neg, jnp.float32)

        segs = []
        for g in range(_G):
            m = bt == g
            segs.append(jnp.max(jnp.where(m, h, neg), axis=0))
        blockmax = jnp.stack(segs, axis=0)
        pooled[...] = jnp.maximum(pooled[...], blockmax)

        @pl.when(i == n_steps - 1)
        def _():
            p = pooled[...]
            p = jnp.where(p == neg, 0.0, p)
            z = jnp.dot(p, fw1_ref[...], preferred_element_type=jnp.float32) + fb1_ref[0]
            z = jnp.maximum(z, 0.0)
            out_ref[...] = (
                jnp.dot(z, fw2_ref[...], preferred_element_type=jnp.float32) + fb2_ref[0]
            )

    return pl.pallas_call(
        body,
        grid=(n_steps,),
        in_specs=[
            pl.BlockSpec((2, _BN, _F), lambda i: (0, i, 0)),
            pl.BlockSpec((2, _BN, _F), lambda i: (0, i, 0)),
            pl.BlockSpec((_BN, 1), lambda i: (i, 0)),
            pl.BlockSpec((1, 256), lambda i: (0, 0)),
            pl.BlockSpec((_BN, 1), lambda i: (i, 0)),
            pl.BlockSpec((256, 512), lambda i: (0, 0)),
            pl.BlockSpec((1, 512), lambda i: (0, 0)),
            pl.BlockSpec((512, 10), lambda i: (0, 0)),
            pl.BlockSpec((1, 10), lambda i: (0, 0)),
        ],
        out_specs=pl.BlockSpec((_G, 10), lambda i: (0, 0)),
        out_shape=jax.ShapeDtypeStruct((_G, 10), jnp.float32),
        scratch_shapes=[pltpu.VMEM((_G, 2 * _F), jnp.float32)],
    )(agg3, y3, dinv, b3, batch2d, FW1, Fb1, FW2, Fb2)


def kernel(x, edge_index, batch, W1, b1, W2, b2, W3, b3, FW1, Fb1, FW2, Fb2):
    src = edge_index[0].astype(jnp.int32)
    dst = edge_index[1].astype(jnp.int32)
    src2 = src.reshape(_E // _EB, _EB)
    dst2 = dst.reshape(_E // _EB, _EB)
    batch2d = batch.astype(jnp.int32).reshape(_N, 1)

    deg32 = _deg_kernel(dst)
    y1, dinv = _y1_call(x, W1, deg32.T)

    agg1 = _agg2(y1.reshape(2 * _N, _F), src2, dst2).reshape(2, _N, _F)
    y2 = _mid_call(agg1, y1, dinv, b1.reshape(1, 256), W2, 2, 4)

    agg2 = _agg4(y2.reshape(4 * _N, _F), src2, dst2).reshape(4, _N, _F)
    y3 = _mid_call(agg2, y2, dinv, b2.reshape(1, 512), W3, 4, 2)

    agg3 = _agg2(y3.reshape(2 * _N, _F), src2, dst2).reshape(2, _N, _F)
    return _head_call(
        agg3, y3, dinv, b3.reshape(1, 256), batch2d,
        FW1, Fb1.reshape(1, 512), FW2, Fb2.reshape(1, 10)
    )


# speedup vs baseline: 14.7097x; 1.1286x over previous
import functools

import jax
import jax.numpy as jnp
from jax import lax
from jax.experimental import pallas as pl
from jax.experimental.pallas import tpu as pltpu
from jax.experimental.pallas import tpu_sc as plsc

_N = 10000
_E = 320000
_G = 64
_F = 128
_NC = 2
_NS = 16
_EB = 80
_EPT = _E // _NS
_NEB = _EPT // _EB
_RB = 40
_NRB = _N // _RB
_BN = 2000


def _sc_mesh():
    return plsc.VectorSubcoreMesh(
        core_axis_name="c", subcore_axis_name="s", num_cores=_NC, num_subcores=_NS
    )


@functools.partial(
    pl.kernel,
    out_type=jax.ShapeDtypeStruct((_NC * _NS, _N), jnp.float32),
    mesh=_sc_mesh(),
    scratch_types=[
        pltpu.VMEM((_N,), jnp.float32),
        pltpu.VMEM((2000,), jnp.int32),
    ],
    compiler_params=pltpu.CompilerParams(needs_layout_passes=False),
)
def _deg_kernel(dst_hbm, out_hbm, deg_v, idx_v):
    core = lax.axis_index("c")
    s = lax.axis_index("s")
    wid = core * _NS + s
    zeros16 = jnp.zeros((16,), jnp.float32)

    def zero_body(i, carry):
        deg_v[pl.ds(i * 16, 16)] = zeros16
        return carry

    lax.fori_loop(0, _N // 16, zero_body, 0)

    ones16 = jnp.ones((16,), jnp.float32)
    ept = _E // (_NC * _NS)

    def blk_body(b, carry):
        pltpu.sync_copy(dst_hbm.at[pl.ds(wid * ept + b * 2000, 2000)], idx_v)

        def inner(j, c2):
            idx = idx_v[pl.ds(j * 16, 16)]
            plsc.addupdate_scatter(deg_v, [idx], ones16)
            return c2

        lax.fori_loop(0, 2000 // 16, inner, 0)
        return carry

    lax.fori_loop(0, ept // 2000, blk_body, 0)
    pltpu.sync_copy(deg_v, out_hbm.at[wid])


_K = 16
_NB = 4
_NGRP = _E // _EB // _K


def _make_agg_kernel(n_chunks):
    cpc = n_chunks // _NC

    @functools.partial(
        pl.kernel,
        out_type=jax.ShapeDtypeStruct((n_chunks * _N, _F), jnp.float32),
        mesh=_sc_mesh(),
        scratch_types=[
            pltpu.VMEM((_K, _EB), jnp.int32),
            pltpu.VMEM((_K, _EB), jnp.int32),
            [pltpu.VMEM((_EB, _F), jnp.float32) for _ in range(_NB)],
            pltpu.VMEM((_RB, _F), jnp.float32),
            pltpu.VMEM_SHARED((_N, _F), jnp.float32),
            pltpu.SemaphoreType.DMA((_NB,)),
            pltpu.SemaphoreType.DMA((_NB,)),
        ],
    )
    def agg_kernel(y_hbm, src_hbm, dst_hbm, out_hbm, src_g, dst_g, gbufs,
                   zbuf, accum, gsem, ssem):
        core = lax.axis_index("c")
        s = lax.axis_index("s")
        zeros16 = jnp.zeros((16,), jnp.float32)

        def zb_body(i, carry):
            r = i // (_F // 16)
            j = i % (_F // 16)
            zbuf[r, pl.ds(j * 16, 16)] = zeros16
            return carry

        lax.fori_loop(0, _RB * (_F // 16), zb_body, 0)

        nrb_mine = (_NRB - s + _NS - 1) // _NS

        for cc in range(cpc):
            chunk = core * cpc + cc
            off = chunk * _N

            def zero_body(k, carry):
                r0 = (s + k * _NS) * _RB
                pltpu.sync_copy(zbuf, accum.at[pl.ds(r0, _RB)])
                return carry

            lax.fori_loop(0, nrb_mine, zero_body, 0)
            plsc.subcore_barrier()

            def edge_body(g, carry):
                row0 = (s + g * _NS) * _K
                pltpu.sync_copy(src_hbm.at[pl.ds(row0, _K)], src_g)
                pltpu.sync_copy(dst_hbm.at[pl.ds(row0, _K)], dst_g)
                for k in range(_K):
                    for j in range(_EB // 16):
                        src_g[k, pl.ds(j * 16, 16)] = (
                            src_g[k, pl.ds(j * 16, 16)] + off
                        )
                gds = {}
                sds = {}
                for k in range(_NB):
                    gds[k] = pltpu.async_copy(
                        y_hbm.at[src_g.at[k]], gbufs[k], gsem.at[k]
                    )
                for k in range(_K):
                    gds[k].wait()
                    if k >= 2:
                        sds[k - 2].wait()
                        if k + 2 < _K:
                            gds[k + 2] = pltpu.async_copy(
                                y_hbm.at[src_g.at[k + 2]],
                                gbufs[(k + 2) % _NB],
                                gsem.at[(k + 2) % _NB],
                            )
                    sds[k] = pltpu.async_copy(
                        gbufs[k % _NB], accum.at[dst_g.at[k]],
                        ssem.at[k % _NB], add=True,
                    )
                sds[_K - 2].wait()
                sds[_K - 1].wait()
                return carry

            ngrp_mine = (_NGRP - s + _NS - 1) // _NS
            lax.fori_loop(0, ngrp_mine, edge_body, 0)
            plsc.subcore_barrier()

            def wr_body(k, carry):
                r0 = (s + k * _NS) * _RB
                pltpu.sync_copy(accum.at[pl.ds(r0, _RB)], out_hbm.at[pl.ds(off + r0, _RB)])
                return carry

            lax.fori_loop(0, nrb_mine, wr_body, 0)
            if cc + 1 < cpc:
                plsc.subcore_barrier()

    return agg_kernel


_agg2 = _make_agg_kernel(2)
_agg4 = _make_agg_kernel(4)


def _y1_call(x, W1, deg32):
    def body(x_ref, w_ref, deg_ref, y_ref, dinv_ref):
        d = jnp.sum(deg_ref[...], axis=1) + 2.0
        dinv = lax.rsqrt(d)[:, None]
        xw = jnp.dot(x_ref[...], w_ref[...], preferred_element_type=jnp.float32)
        y = xw * dinv
        y_ref[0] = y[:, :_F]
        y_ref[1] = y[:, _F:]
        dinv_ref[...] = dinv

    return pl.pallas_call(
        body,
        grid=(_N // _BN,),
        in_specs=[
            pl.BlockSpec((_BN, 128), lambda i: (i, 0)),
            pl.BlockSpec((128, 256), lambda i: (0, 0)),
            pl.BlockSpec((_BN, _NC * _NS), lambda i: (i, 0)),
        ],
        out_specs=[
            pl.BlockSpec((2, _BN, _F), lambda i: (0, i, 0)),
            pl.BlockSpec((_BN, 1), lambda i: (i, 0)),
        ],
        out_shape=[
            jax.ShapeDtypeStruct((2, _N, _F), jnp.float32),
            jax.ShapeDtypeStruct((_N, 1), jnp.float32),
        ],
    )(x, W1, deg32)


def _mid_call(agg, y, dinv, b, W, c_in, c_out):
    d_in = c_in * _F
    d_out = c_out * _F

    def body(agg_ref, y_ref, dinv_ref, b_ref, w_ref, out_ref):
        dv = dinv_ref[...]
        acc = jnp.zeros((_BN, d_out), jnp.float32)
        for c in range(c_in):
            h = dv * (agg_ref[c] + 2.0 * y_ref[c]) + b_ref[0, c * _F:(c + 1) * _F]
            h = jnp.maximum(h, 0.0)
            acc = acc + jnp.dot(
                h, w_ref[c * _F:(c + 1) * _F, :], preferred_element_type=jnp.float32
            )
        y_next = acc * dv
        for co in range(c_out):
            out_ref[co] = y_next[:, co * _F:(co + 1) * _F]

    return pl.pallas_call(
        body,
        grid=(_N // _BN,),
        in_specs=[
            pl.BlockSpec((c_in, _BN, _F), lambda i: (0, i, 0)),
            pl.BlockSpec((c_in, _BN, _F), lambda i: (0, i, 0)),
            pl.BlockSpec((_BN, 1), lambda i: (i, 0)),
            pl.BlockSpec((1, d_in), lambda i: (0, 0)),
            pl.BlockSpec((d_in, d_out), lambda i: (0, 0)),
        ],
        out_specs=pl.BlockSpec((c_out, _BN, _F), lambda i: (0, i, 0)),
        out_shape=jax.ShapeDtypeStruct((c_out, _N, _F), jnp.float32),
    )(agg, y, dinv, b, W)


def _head_call(agg3, y3, dinv, b3, batch2d, FW1, Fb1, FW2, Fb2):
    n_steps = _N // _BN

    def body(agg_ref, y_ref, dinv_ref, b_ref, bt_ref, fw1_ref, fb1_ref,
             fw2_ref, fb2_ref, out_ref, pooled):
        i = pl.program_id(0)
        dv = dinv_ref[...]
        h0 = dv * (agg_ref[0] + 2.0 * y_ref[0]) + b_ref[0, :_F]
        h1 = dv * (agg_ref[1] + 2.0 * y_ref[1]) + b_ref[0, _F:]
        h = jnp.concatenate([h0, h1], axis=1)
        bt = bt_ref[...]
        neg = jnp.float32(-jnp.inf)

        @pl.when(i == 0)
        def _():
            pooled[...] = jnp.full((_G, 2 * _F), neg, jnp.float32)

        segs = []
        for g in range(_G):
            m = bt == g
            segs.append(jnp.max(jnp.where(m, h, neg), axis=0))
        blockmax = jnp.stack(segs, axis=0)
        pooled[...] = jnp.maximum(pooled[...], blockmax)

        @pl.when(i == n_steps - 1)
        def _():
            p = pooled[...]
            p = jnp.where(p == neg, 0.0, p)
            z = jnp.dot(p, fw1_ref[...], preferred_element_type=jnp.float32) + fb1_ref[0]
            z = jnp.maximum(z, 0.0)
            out_ref[...] = (
                jnp.dot(z, fw2_ref[...], preferred_element_type=jnp.float32) + fb2_ref[0]
            )

    return pl.pallas_call(
        body,
        grid=(n_steps,),
        in_specs=[
            pl.BlockSpec((2, _BN, _F), lambda i: (0, i, 0)),
            pl.BlockSpec((2, _BN, _F), lambda i: (0, i, 0)),
            pl.BlockSpec((_BN, 1), lambda i: (i, 0)),
            pl.BlockSpec((1, 256), lambda i: (0, 0)),
            pl.BlockSpec((_BN, 1), lambda i: (i, 0)),
            pl.BlockSpec((256, 512), lambda i: (0, 0)),
            pl.BlockSpec((1, 512), lambda i: (0, 0)),
            pl.BlockSpec((512, 10), lambda i: (0, 0)),
            pl.BlockSpec((1, 10), lambda i: (0, 0)),
        ],
        out_specs=pl.BlockSpec((_G, 10), lambda i: (0, 0)),
        out_shape=jax.ShapeDtypeStruct((_G, 10), jnp.float32),
        scratch_shapes=[pltpu.VMEM((_G, 2 * _F), jnp.float32)],
    )(agg3, y3, dinv, b3, batch2d, FW1, Fb1, FW2, Fb2)


def kernel(x, edge_index, batch, W1, b1, W2, b2, W3, b3, FW1, Fb1, FW2, Fb2):
    src = edge_index[0].astype(jnp.int32)
    dst = edge_index[1].astype(jnp.int32)
    src2 = src.reshape(_E // _EB, _EB)
    dst2 = dst.reshape(_E // _EB, _EB)
    batch2d = batch.astype(jnp.int32).reshape(_N, 1)

    deg32 = _deg_kernel(dst)
    y1, dinv = _y1_call(x, W1, deg32.T)

    agg1 = _agg2(y1.reshape(2 * _N, _F), src2, dst2).reshape(2, _N, _F)
    y2 = _mid_call(agg1, y1, dinv, b1.reshape(1, 256), W2, 2, 4)

    agg2 = _agg4(y2.reshape(4 * _N, _F), src2, dst2).reshape(4, _N, _F)
    y3 = _mid_call(agg2, y2, dinv, b2.reshape(1, 512), W3, 4, 2)

    agg3 = _agg2(y3.reshape(2 * _N, _F), src2, dst2).reshape(2, _N, _F)
    return _head_call(
        agg3, y3, dinv, b3.reshape(1, 256), batch2d,
        FW1, Fb1.reshape(1, 512), FW2, Fb2.reshape(1, 10)
    )


# speedup vs baseline: 15.3694x; 1.0448x over previous
import functools

import jax
import jax.numpy as jnp
from jax import lax
from jax.experimental import pallas as pl
from jax.experimental.pallas import tpu as pltpu
from jax.experimental.pallas import tpu_sc as plsc

_N = 10000
_E = 320000
_G = 64
_F = 128
_NC = 2
_NS = 16
_EB = 80
_EPT = _E // _NS
_NEB = _EPT // _EB
_RB = 40
_NRB = _N // _RB
_RBW = 200
_BN = 2000


def _sc_mesh():
    return plsc.VectorSubcoreMesh(
        core_axis_name="c", subcore_axis_name="s", num_cores=_NC, num_subcores=_NS
    )


@functools.partial(
    pl.kernel,
    out_type=jax.ShapeDtypeStruct((_NC * _NS, _N), jnp.float32),
    mesh=_sc_mesh(),
    scratch_types=[
        pltpu.VMEM((_N,), jnp.float32),
        pltpu.VMEM((2000,), jnp.int32),
    ],
    compiler_params=pltpu.CompilerParams(needs_layout_passes=False),
)
def _deg_kernel(dst_hbm, out_hbm, deg_v, idx_v):
    core = lax.axis_index("c")
    s = lax.axis_index("s")
    wid = core * _NS + s
    zeros16 = jnp.zeros((16,), jnp.float32)

    def zero_body(i, carry):
        deg_v[pl.ds(i * 16, 16)] = zeros16
        return carry

    lax.fori_loop(0, _N // 16, zero_body, 0)

    ones16 = jnp.ones((16,), jnp.float32)
    ept = _E // (_NC * _NS)

    def blk_body(b, carry):
        pltpu.sync_copy(dst_hbm.at[pl.ds(wid * ept + b * 2000, 2000)], idx_v)

        def inner(j, c2):
            idx = idx_v[pl.ds(j * 16, 16)]
            plsc.addupdate_scatter(deg_v, [idx], ones16)
            return c2

        lax.fori_loop(0, 2000 // 16, inner, 0)
        return carry

    lax.fori_loop(0, ept // 2000, blk_body, 0)
    pltpu.sync_copy(deg_v, out_hbm.at[wid])


_K = 16
_NB = 4
_NGRP = _E // _EB // _K


def _make_agg_kernel(n_chunks):
    cpc = n_chunks // _NC

    @functools.partial(
        pl.kernel,
        out_type=jax.ShapeDtypeStruct((n_chunks * _N, _F), jnp.float32),
        mesh=_sc_mesh(),
        scratch_types=[
            pltpu.VMEM((_K, _EB), jnp.int32),
            pltpu.VMEM((_K, _EB), jnp.int32),
            pltpu.VMEM((2, _EB), jnp.int32),
            [pltpu.VMEM((_EB, _F), jnp.float32) for _ in range(_NB)],
            pltpu.VMEM((_RB, _F), jnp.float32),
            pltpu.VMEM_SHARED((_N, _F), jnp.float32),
            pltpu.SemaphoreType.DMA((_NB,)),
            pltpu.SemaphoreType.DMA((_NB + 2,)),
        ],
    )
    def agg_kernel(y_hbm, src_hbm, dst_hbm, out_hbm, src_g, dst_g, tdst, gbufs,
                   zbuf, accum, gsem, ssem):
        core = lax.axis_index("c")
        s = lax.axis_index("s")
        zeros16 = jnp.zeros((16,), jnp.float32)

        def zb_body(i, carry):
            r = i // (_F // 16)
            j = i % (_F // 16)
            zbuf[r, pl.ds(j * 16, 16)] = zeros16
            return carry

        lax.fori_loop(0, _RB * (_F // 16), zb_body, 0)

        nrb_mine = (_NRB - s + _NS - 1) // _NS

        for cc in range(cpc):
            chunk = core * cpc + cc
            off = chunk * _N

            def zero_body(k, carry):
                r0 = (s + k * _NS) * _RB
                pltpu.sync_copy(zbuf, accum.at[pl.ds(r0, _RB)])
                return carry

            lax.fori_loop(0, nrb_mine, zero_body, 0)
            plsc.subcore_barrier()

            def drain_tails():
                pltpu.make_async_copy(
                    gbufs[2], accum.at[tdst.at[0]], ssem.at[_NB]
                ).wait()
                pltpu.make_async_copy(
                    gbufs[3], accum.at[tdst.at[1]], ssem.at[_NB + 1]
                ).wait()

            def edge_body(g, carry):
                row0 = (s + g * _NS) * _K
                pltpu.sync_copy(src_hbm.at[pl.ds(row0, _K)], src_g)
                pltpu.sync_copy(dst_hbm.at[pl.ds(row0, _K)], dst_g)
                for k in range(_K):
                    for j in range(_EB // 16):
                        src_g[k, pl.ds(j * 16, 16)] = (
                            src_g[k, pl.ds(j * 16, 16)] + off
                        )
                gds = {}
                sds = {}
                for k in range(2):
                    gds[k] = pltpu.async_copy(
                        y_hbm.at[src_g.at[k]], gbufs[k], gsem.at[k]
                    )

                @pl.when(g > 0)
                def _():
                    drain_tails()

                for k in range(2, _NB):
                    gds[k] = pltpu.async_copy(
                        y_hbm.at[src_g.at[k]], gbufs[k], gsem.at[k]
                    )
                for k in range(_K):
                    gds[k].wait()
                    if k >= 2:
                        sds[k - 2].wait()
                        if k + 2 < _K:
                            gds[k + 2] = pltpu.async_copy(
                                y_hbm.at[src_g.at[k + 2]],
                                gbufs[(k + 2) % _NB],
                                gsem.at[(k + 2) % _NB],
                            )
                    if k >= _K - 2:
                        t = k - (_K - 2)
                        for j in range(_EB // 16):
                            tdst[t, pl.ds(j * 16, 16)] = dst_g[k, pl.ds(j * 16, 16)]
                        sds[k] = pltpu.async_copy(
                            gbufs[k % _NB], accum.at[tdst.at[t]],
                            ssem.at[_NB + t], add=True,
                        )
                    else:
                        sds[k] = pltpu.async_copy(
                            gbufs[k % _NB], accum.at[dst_g.at[k]],
                            ssem.at[k % _NB], add=True,
                        )
                return carry

            ngrp_mine = (_NGRP - s + _NS - 1) // _NS
            lax.fori_loop(0, ngrp_mine, edge_body, 0)
            drain_tails()
            plsc.subcore_barrier()

            def wr_body(k, carry):
                r0 = (s + k * _NS) * _RBW
                pltpu.sync_copy(accum.at[pl.ds(r0, _RBW)], out_hbm.at[pl.ds(off + r0, _RBW)])
                return carry

            nwb_mine = (_N // _RBW - s + _NS - 1) // _NS
            lax.fori_loop(0, nwb_mine, wr_body, 0)
            if cc + 1 < cpc:
                plsc.subcore_barrier()

    return agg_kernel


_agg2 = _make_agg_kernel(2)
_agg4 = _make_agg_kernel(4)


def _y1_call(x, W1, deg32):
    def body(x_ref, w_ref, deg_ref, y_ref, dinv_ref):
        d = jnp.sum(deg_ref[...], axis=1) + 2.0
        dinv = lax.rsqrt(d)[:, None]
        xw = jnp.dot(x_ref[...], w_ref[...], preferred_element_type=jnp.float32)
        y = xw * dinv
        y_ref[0] = y[:, :_F]
        y_ref[1] = y[:, _F:]
        dinv_ref[...] = dinv

    return pl.pallas_call(
        body,
        grid=(_N // _BN,),
        in_specs=[
            pl.BlockSpec((_BN, 128), lambda i: (i, 0)),
            pl.BlockSpec((128, 256), lambda i: (0, 0)),
            pl.BlockSpec((_BN, _NC * _NS), lambda i: (i, 0)),
        ],
        out_specs=[
            pl.BlockSpec((2, _BN, _F), lambda i: (0, i, 0)),
            pl.BlockSpec((_BN, 1), lambda i: (i, 0)),
        ],
        out_shape=[
            jax.ShapeDtypeStruct((2, _N, _F), jnp.float32),
            jax.ShapeDtypeStruct((_N, 1), jnp.float32),
        ],
    )(x, W1, deg32)


def _mid_call(agg, y, dinv, b, W, c_in, c_out):
    d_in = c_in * _F
    d_out = c_out * _F

    def body(agg_ref, y_ref, dinv_ref, b_ref, w_ref, out_ref):
        dv = dinv_ref[...]
        acc = jnp.zeros((_BN, d_out), jnp.float32)
        for c in range(c_in):
            h = dv * (agg_ref[c] + 2.0 * y_ref[c]) + b_ref[0, c * _F:(c + 1) * _F]
            h = jnp.maximum(h, 0.0)
            acc = acc + jnp.dot(
                h, w_ref[c * _F:(c + 1) * _F, :], preferred_element_type=jnp.float32
            )
        y_next = acc * dv
        for co in range(c_out):
            out_ref[co] = y_next[:, co * _F:(co + 1) * _F]

    return pl.pallas_call(
        body,
        grid=(_N // _BN,),
        in_specs=[
            pl.BlockSpec((c_in, _BN, _F), lambda i: (0, i, 0)),
            pl.BlockSpec((c_in, _BN, _F), lambda i: (0, i, 0)),
            pl.BlockSpec((_BN, 1), lambda i: (i, 0)),
            pl.BlockSpec((1, d_in), lambda i: (0, 0)),
            pl.BlockSpec((d_in, d_out), lambda i: (0, 0)),
        ],
        out_specs=pl.BlockSpec((c_out, _BN, _F), lambda i: (0, i, 0)),
        out_shape=jax.ShapeDtypeStruct((c_out, _N, _F), jnp.float32),
    )(agg, y, dinv, b, W)


def _head_call(agg3, y3, dinv, b3, batch2d, FW1, Fb1, FW2, Fb2):
    n_steps = _N // _BN

    def body(agg_ref, y_ref, dinv_ref, b_ref, bt_ref, fw1_ref, fb1_ref,
             fw2_ref, fb2_ref, out_ref, pooled):
        i = pl.program_id(0)
        dv = dinv_ref[...]
        h0 = dv * (agg_ref[0] + 2.0 * y_ref[0]) + b_ref[0, :_F]
        h1 = dv * (agg_ref[1] + 2.0 * y_ref[1]) + b_ref[0, _F:]
        h = jnp.concatenate([h0, h1], axis=1)
        bt = bt_ref[...]
        neg = jnp.float32(-jnp.inf)

        @pl.when(i == 0)
        def _():
            pooled[...] = jnp.full((_G, 2 * _F), neg, jnp.float32)

        segs = []
        for g in range(_G):
            m = bt == g
            segs.append(jnp.max(jnp.where(m, h, neg), axis=0))
        blockmax = jnp.stack(segs, axis=0)
        pooled[...] = jnp.maximum(pooled[...], blockmax)

        @pl.when(i == n_steps - 1)
        def _():
            p = pooled[...]
            p = jnp.where(p == neg, 0.0, p)
            z = jnp.dot(p, fw1_ref[...], preferred_element_type=jnp.float32) + fb1_ref[0]
            z = jnp.maximum(z, 0.0)
            out_ref[...] = (
                jnp.dot(z, fw2_ref[...], preferred_element_type=jnp.float32) + fb2_ref[0]
            )

    return pl.pallas_call(
        body,
        grid=(n_steps,),
        in_specs=[
            pl.BlockSpec((2, _BN, _F), lambda i: (0, i, 0)),
            pl.BlockSpec((2, _BN, _F), lambda i: (0, i, 0)),
            pl.BlockSpec((_BN, 1), lambda i: (i, 0)),
            pl.BlockSpec((1, 256), lambda i: (0, 0)),
            pl.BlockSpec((_BN, 1), lambda i: (i, 0)),
            pl.BlockSpec((256, 512), lambda i: (0, 0)),
            pl.BlockSpec((1, 512), lambda i: (0, 0)),
            pl.BlockSpec((512, 10), lambda i: (0, 0)),
            pl.BlockSpec((1, 10), lambda i: (0, 0)),
        ],
        out_specs=pl.BlockSpec((_G, 10), lambda i: (0, 0)),
        out_shape=jax.ShapeDtypeStruct((_G, 10), jnp.float32),
        scratch_shapes=[pltpu.VMEM((_G, 2 * _F), jnp.float32)],
    )(agg3, y3, dinv, b3, batch2d, FW1, Fb1, FW2, Fb2)


def kernel(x, edge_index, batch, W1, b1, W2, b2, W3, b3, FW1, Fb1, FW2, Fb2):
    src = edge_index[0].astype(jnp.int32)
    dst = edge_index[1].astype(jnp.int32)
    src2 = src.reshape(_E // _EB, _EB)
    dst2 = dst.reshape(_E // _EB, _EB)
    batch2d = batch.astype(jnp.int32).reshape(_N, 1)

    deg32 = _deg_kernel(dst)
    y1, dinv = _y1_call(x, W1, deg32.T)

    agg1 = _agg2(y1.reshape(2 * _N, _F), src2, dst2).reshape(2, _N, _F)
    y2 = _mid_call(agg1, y1, dinv, b1.reshape(1, 256), W2, 2, 4)

    agg2 = _agg4(y2.reshape(4 * _N, _F), src2, dst2).reshape(4, _N, _F)
    y3 = _mid_call(agg2, y2, dinv, b2.reshape(1, 512), W3, 4, 2)

    agg3 = _agg2(y3.reshape(2 * _N, _F), src2, dst2).reshape(2, _N, _F)
    return _head_call(
        agg3, y3, dinv, b3.reshape(1, 256), batch2d,
        FW1, Fb1.reshape(1, 512), FW2, Fb2.reshape(1, 10)
    )


# speedup vs baseline: 16.8724x; 1.0978x over previous
import functools

import jax
import jax.numpy as jnp
from jax import lax
from jax.experimental import pallas as pl
from jax.experimental.pallas import tpu as pltpu
from jax.experimental.pallas import tpu_sc as plsc

_N = 10000
_E = 320000
_G = 64
_F = 128
_NC = 2
_NS = 16
_EB = 80
_EPT = _E // _NS
_NEB = _EPT // _EB
_RB = 40
_NRB = _N // _RB
_RBW = 200
_BN = 2000


def _sc_mesh():
    return plsc.VectorSubcoreMesh(
        core_axis_name="c", subcore_axis_name="s", num_cores=_NC, num_subcores=_NS
    )


@functools.partial(
    pl.kernel,
    out_type=jax.ShapeDtypeStruct((_NC * _NS, _N), jnp.float32),
    mesh=_sc_mesh(),
    scratch_types=[
        pltpu.VMEM((_N,), jnp.float32),
        pltpu.VMEM((2000,), jnp.int32),
    ],
    compiler_params=pltpu.CompilerParams(needs_layout_passes=False),
)
def _deg_kernel(dst_hbm, out_hbm, deg_v, idx_v):
    core = lax.axis_index("c")
    s = lax.axis_index("s")
    wid = core * _NS + s
    zeros16 = jnp.zeros((16,), jnp.float32)

    def zero_body(i, carry):
        deg_v[pl.ds(i * 16, 16)] = zeros16
        return carry

    lax.fori_loop(0, _N // 16, zero_body, 0)

    ones16 = jnp.ones((16,), jnp.float32)
    ept = _E // (_NC * _NS)

    def blk_body(b, carry):
        pltpu.sync_copy(dst_hbm.at[pl.ds(wid * ept + b * 2000, 2000)], idx_v)

        def inner(j, c2):
            idx = idx_v[pl.ds(j * 16, 16)]
            plsc.addupdate_scatter(deg_v, [idx], ones16)
            return c2

        lax.fori_loop(0, 2000 // 16, inner, 0)
        return carry

    lax.fori_loop(0, ept // 2000, blk_body, 0)
    pltpu.sync_copy(deg_v, out_hbm.at[wid])


_K = 8
_NB = 4
_NGRP = _E // _EB // _K


def _make_agg_kernel(n_chunks):
    cpc = n_chunks // _NC

    @functools.partial(
        pl.kernel,
        out_type=jax.ShapeDtypeStruct((n_chunks * _N, _F), jnp.float32),
        mesh=_sc_mesh(),
        scratch_types=[
            [pltpu.VMEM((_K, _EB), jnp.int32) for _ in range(2)],
            [pltpu.VMEM((_K, _EB), jnp.int32) for _ in range(2)],
            pltpu.VMEM((2, _EB), jnp.int32),
            [pltpu.VMEM((_EB, _F), jnp.float32) for _ in range(_NB)],
            pltpu.VMEM((_RB, _F), jnp.float32),
            pltpu.VMEM_SHARED((_N, _F), jnp.float32),
            pltpu.SemaphoreType.DMA((_NB,)),
            pltpu.SemaphoreType.DMA((_NB + 2,)),
            pltpu.SemaphoreType.DMA((2,)),
        ],
    )
    def agg_kernel(y_hbm, src_hbm, dst_hbm, out_hbm, src_gs, dst_gs, tdst, gbufs,
                   zbuf, accum, gsem, ssem, isem):
        core = lax.axis_index("c")
        s = lax.axis_index("s")
        zeros16 = jnp.zeros((16,), jnp.float32)

        def zb_body(i, carry):
            r = i // (_F // 16)
            j = i % (_F // 16)
            zbuf[r, pl.ds(j * 16, 16)] = zeros16
            return carry

        lax.fori_loop(0, _RB * (_F // 16), zb_body, 0)

        nrb_mine = (_NRB - s + _NS - 1) // _NS

        for cc in range(cpc):
            chunk = core * cpc + cc
            off = chunk * _N

            def zero_body(k, carry):
                r0 = (s + k * _NS) * _RB
                pltpu.sync_copy(zbuf, accum.at[pl.ds(r0, _RB)])
                return carry

            lax.fori_loop(0, nrb_mine, zero_body, 0)
            plsc.subcore_barrier()

            def drain_tails():
                pltpu.make_async_copy(
                    gbufs[2], accum.at[tdst.at[0]], ssem.at[_NB]
                ).wait()
                pltpu.make_async_copy(
                    gbufs[3], accum.at[tdst.at[1]], ssem.at[_NB + 1]
                ).wait()

            def load_idx_sync(g, sg, dg):
                row0 = (s + g * _NS) * _K
                pltpu.sync_copy(src_hbm.at[pl.ds(row0, _K)], sg)
                pltpu.sync_copy(dst_hbm.at[pl.ds(row0, _K)], dg)

            def prefetch_idx(g, sg, dg, sem):
                row0 = (s + g * _NS) * _K
                pltpu.async_copy(src_hbm.at[pl.ds(row0, _K)], sg, sem)
                pltpu.async_copy(dst_hbm.at[pl.ds(row0, _K)], dg, sem)

            def wait_idx(sg, dg, sem):
                pltpu.make_async_copy(src_hbm.at[pl.ds(0, _K)], sg, sem).wait()
                pltpu.make_async_copy(dst_hbm.at[pl.ds(0, _K)], dg, sem).wait()

            def process_group(g, src_g, dst_g, drain_pred, pf_pred, par):
                for k in range(_K):
                    for j in range(_EB // 16):
                        src_g[k, pl.ds(j * 16, 16)] = (
                            src_g[k, pl.ds(j * 16, 16)] + off
                        )
                gds = {}
                sds = {}
                for k in range(2):
                    gds[k] = pltpu.async_copy(
                        y_hbm.at[src_g.at[k]], gbufs[k], gsem.at[k]
                    )

                @pl.when(drain_pred)
                def _():
                    drain_tails()

                for k in range(2, _NB):
                    gds[k] = pltpu.async_copy(
                        y_hbm.at[src_g.at[k]], gbufs[k], gsem.at[k]
                    )

                if pf_pred is not None:
                    @pl.when(pf_pred)
                    def _():
                        prefetch_idx(
                            g + 1, src_gs[1 - par], dst_gs[1 - par],
                            isem.at[1 - par],
                        )

                for k in range(_K):
                    gds[k].wait()
                    if k >= 2:
                        sds[k - 2].wait()
                        if k + 2 < _K:
                            gds[k + 2] = pltpu.async_copy(
                                y_hbm.at[src_g.at[k + 2]],
                                gbufs[(k + 2) % _NB],
                                gsem.at[(k + 2) % _NB],
                            )
                    if k >= _K - 2:
                        t = k - (_K - 2)
                        for j in range(_EB // 16):
                            tdst[t, pl.ds(j * 16, 16)] = dst_g[k, pl.ds(j * 16, 16)]
                        sds[k] = pltpu.async_copy(
                            gbufs[k % _NB], accum.at[tdst.at[t]],
                            ssem.at[_NB + t], add=True,
                        )
                    else:
                        sds[k] = pltpu.async_copy(
                            gbufs[k % _NB], accum.at[dst_g.at[k]],
                            ssem.at[k % _NB], add=True,
                        )

            ngrp_mine = (_NGRP - s + _NS - 1) // _NS
            npairs = ngrp_mine // 2
            rem = ngrp_mine - 2 * npairs

            load_idx_sync(0, src_gs[0], dst_gs[0])

            def pair_body(p, carry):
                ga = 2 * p
                gb = 2 * p + 1

                @pl.when(p > 0)
                def _():
                    wait_idx(src_gs[0], dst_gs[0], isem.at[0])

                process_group(ga, src_gs[0], dst_gs[0], p > 0,
                              gb < ngrp_mine, 0)

                wait_idx(src_gs[1], dst_gs[1], isem.at[1])
                process_group(gb, src_gs[1], dst_gs[1], ga >= 0,
                              gb + 1 < ngrp_mine, 1)
                return carry

            lax.fori_loop(0, npairs, pair_body, 0)

            @pl.when(rem == 1)
            def _():
                g_last = 2 * npairs

                @pl.when(npairs > 0)
                def _():
                    wait_idx(src_gs[0], dst_gs[0], isem.at[0])

                process_group(g_last, src_gs[0], dst_gs[0], npairs > 0,
                              None, 0)

            drain_tails()
            plsc.subcore_barrier()

            def wr_body(k, carry):
                r0 = (s + k * _NS) * _RBW
                pltpu.sync_copy(accum.at[pl.ds(r0, _RBW)], out_hbm.at[pl.ds(off + r0, _RBW)])
                return carry

            nwb_mine = (_N // _RBW - s + _NS - 1) // _NS
            lax.fori_loop(0, nwb_mine, wr_body, 0)
            if cc + 1 < cpc:
                plsc.subcore_barrier()

    return agg_kernel


_agg2 = _make_agg_kernel(2)
_agg4 = _make_agg_kernel(4)


def _y1_call(x, W1, deg32):
    def body(x_ref, w_ref, deg_ref, y_ref, dinv_ref):
        d = jnp.sum(deg_ref[...], axis=1) + 2.0
        dinv = lax.rsqrt(d)[:, None]
        xw = jnp.dot(x_ref[...], w_ref[...], preferred_element_type=jnp.float32)
        y = xw * dinv
        y_ref[0] = y[:, :_F]
        y_ref[1] = y[:, _F:]
        dinv_ref[...] = dinv

    return pl.pallas_call(
        body,
        grid=(_N // _BN,),
        in_specs=[
            pl.BlockSpec((_BN, 128), lambda i: (i, 0)),
            pl.BlockSpec((128, 256), lambda i: (0, 0)),
            pl.BlockSpec((_BN, _NC * _NS), lambda i: (i, 0)),
        ],
        out_specs=[
            pl.BlockSpec((2, _BN, _F), lambda i: (0, i, 0)),
            pl.BlockSpec((_BN, 1), lambda i: (i, 0)),
        ],
        out_shape=[
            jax.ShapeDtypeStruct((2, _N, _F), jnp.float32),
            jax.ShapeDtypeStruct((_N, 1), jnp.float32),
        ],
    )(x, W1, deg32)


def _mid_call(agg, y, dinv, b, W, c_in, c_out):
    d_in = c_in * _F
    d_out = c_out * _F

    def body(agg_ref, y_ref, dinv_ref, b_ref, w_ref, out_ref):
        dv = dinv_ref[...]
        acc = jnp.zeros((_BN, d_out), jnp.float32)
        for c in range(c_in):
            h = dv * (agg_ref[c] + 2.0 * y_ref[c]) + b_ref[0, c * _F:(c + 1) * _F]
            h = jnp.maximum(h, 0.0)
            acc = acc + jnp.dot(
                h, w_ref[c * _F:(c + 1) * _F, :], preferred_element_type=jnp.float32
            )
        y_next = acc * dv
        for co in range(c_out):
            out_ref[co] = y_next[:, co * _F:(co + 1) * _F]

    return pl.pallas_call(
        body,
        grid=(_N // _BN,),
        in_specs=[
            pl.BlockSpec((c_in, _BN, _F), lambda i: (0, i, 0)),
            pl.BlockSpec((c_in, _BN, _F), lambda i: (0, i, 0)),
            pl.BlockSpec((_BN, 1), lambda i: (i, 0)),
            pl.BlockSpec((1, d_in), lambda i: (0, 0)),
            pl.BlockSpec((d_in, d_out), lambda i: (0, 0)),
        ],
        out_specs=pl.BlockSpec((c_out, _BN, _F), lambda i: (0, i, 0)),
        out_shape=jax.ShapeDtypeStruct((c_out, _N, _F), jnp.float32),
    )(agg, y, dinv, b, W)


def _head_call(agg3, y3, dinv, b3, batch2d, FW1, Fb1, FW2, Fb2):
    n_steps = _N // _BN

    def body(agg_ref, y_ref, dinv_ref, b_ref, bt_ref, fw1_ref, fb1_ref,
             fw2_ref, fb2_ref, out_ref, pooled):
        i = pl.program_id(0)
        dv = dinv_ref[...]
        h0 = dv * (agg_ref[0] + 2.0 * y_ref[0]) + b_ref[0, :_F]
        h1 = dv * (agg_ref[1] + 2.0 * y_ref[1]) + b_ref[0, _F:]
        h = jnp.concatenate([h0, h1], axis=1)
        bt = bt_ref[...]
        neg = jnp.float32(-jnp.inf)

        @pl.when(i == 0)
        def _():
            pooled[...] = jnp.full((_G, 2 * _F), neg, jnp.float32)

        segs = []
        for g in range(_G):
            m = bt == g
            segs.append(jnp.max(jnp.where(m, h, neg), axis=0))
        blockmax = jnp.stack(segs, axis=0)
        pooled[...] = jnp.maximum(pooled[...], blockmax)

        @pl.when(i == n_steps - 1)
        def _():
            p = pooled[...]
            p = jnp.where(p == neg, 0.0, p)
            z = jnp.dot(p, fw1_ref[...], preferred_element_type=jnp.float32) + fb1_ref[0]
            z = jnp.maximum(z, 0.0)
            out_ref[...] = (
                jnp.dot(z, fw2_ref[...], preferred_element_type=jnp.float32) + fb2_ref[0]
            )

    return pl.pallas_call(
        body,
        grid=(n_steps,),
        in_specs=[
            pl.BlockSpec((2, _BN, _F), lambda i: (0, i, 0)),
            pl.BlockSpec((2, _BN, _F), lambda i: (0, i, 0)),
            pl.BlockSpec((_BN, 1), lambda i: (i, 0)),
            pl.BlockSpec((1, 256), lambda i: (0, 0)),
            pl.BlockSpec((_BN, 1), lambda i: (i, 0)),
            pl.BlockSpec((256, 512), lambda i: (0, 0)),
            pl.BlockSpec((1, 512), lambda i: (0, 0)),
            pl.BlockSpec((512, 10), lambda i: (0, 0)),
            pl.BlockSpec((1, 10), lambda i: (0, 0)),
        ],
        out_specs=pl.BlockSpec((_G, 10), lambda i: (0, 0)),
        out_shape=jax.ShapeDtypeStruct((_G, 10), jnp.float32),
        scratch_shapes=[pltpu.VMEM((_G, 2 * _F), jnp.float32)],
    )(agg3, y3, dinv, b3, batch2d, FW1, Fb1, FW2, Fb2)


def kernel(x, edge_index, batch, W1, b1, W2, b2, W3, b3, FW1, Fb1, FW2, Fb2):
    src = edge_index[0].astype(jnp.int32)
    dst = edge_index[1].astype(jnp.int32)
    src2 = src.reshape(_E // _EB, _EB)
    dst2 = dst.reshape(_E // _EB, _EB)
    batch2d = batch.astype(jnp.int32).reshape(_N, 1)

    deg32 = _deg_kernel(dst)
    y1, dinv = _y1_call(x, W1, deg32.T)

    agg1 = _agg2(y1.reshape(2 * _N, _F), src2, dst2).reshape(2, _N, _F)
    y2 = _mid_call(agg1, y1, dinv, b1.reshape(1, 256), W2, 2, 4)

    agg2 = _agg4(y2.reshape(4 * _N, _F), src2, dst2).reshape(4, _N, _F)
    y3 = _mid_call(agg2, y2, dinv, b2.reshape(1, 512), W3, 4, 2)

    agg3 = _agg2(y3.reshape(2 * _N, _F), src2, dst2).reshape(2, _N, _F)
    return _head_call(
        agg3, y3, dinv, b3.reshape(1, 256), batch2d,
        FW1, Fb1.reshape(1, 512), FW2, Fb2.reshape(1, 10)
    )
